# baseline (speedup 1.0000x reference)
import jax
import jax.numpy as jnp
from jax import lax
from jax.experimental import pallas as pl
from jax.experimental.pallas import tpu as pltpu
from jax.experimental.pallas import tpu_sc as plsc

_L = 16
_BW = 128
_NW = 32


def kernel(t, embeddings):
    B, H = t.shape
    V, D = embeddings.shape
    n_blocks = H * (B // _BW)
    steps = n_blocks // _NW
    assert n_blocks % _NW == 0 and steps % 2 == 0 and steps >= 4
    assert D % 8 == 0 and V % 2 == 0

    tT = t.T.astype(jnp.int32)
    table2 = jnp.concatenate([embeddings, embeddings], axis=1)
    b_tiles = B // _BW
    G = _BW // _L

    mesh = plsc.VectorSubcoreMesh(core_axis_name="core", subcore_axis_name="subcore")

    @pl.kernel(
        out_type=jax.ShapeDtypeStruct((H * D, B), embeddings.dtype),
        mesh=mesh,
        compiler_params=pltpu.CompilerParams(
            use_tc_tiling_on_sc=True, needs_layout_passes=False
        ),
        scratch_types=[
            pltpu.VMEM((2, _BW), jnp.int32),
            pltpu.VMEM((2, _BW), jnp.int32),
            pltpu.VMEM((2 * _BW, 2 * D), jnp.float32),
            pltpu.VMEM((2, D, _BW + 1), jnp.float32),
            pltpu.SemaphoreType.DMA,
            pltpu.SemaphoreType.DMA,
            pltpu.SemaphoreType.DMA,
            pltpu.SemaphoreType.DMA,
            pltpu.SemaphoreType.DMA,
            pltpu.SemaphoreType.DMA,
        ],
    )
    def gather_kernel(tab_hbm, idx_hbm, o_hbm, raw_v, gidx_v, rows_v,
                      outt_v, si0, si1, sg0, sg1, so0, so1):
        w = lax.axis_index("subcore") * 2 + lax.axis_index("core")
        base = w * steps
        sis = (si0, si1)
        sgs = (sg0, sg1)
        sos = (so0, so1)

        def coords(s):
            blk = base + s
            return blk // b_tiles, (blk % b_tiles) * _BW

        def idx_copy(s, slot):
            h, b0 = coords(s)
            return pltpu.make_async_copy(
                idx_hbm.at[pl.ds(h, 1), pl.ds(b0, _BW)],
                raw_v.at[pl.ds(slot, 1)], sis[slot],
            )

        def gather_copy(slot):
            return pltpu.make_async_copy(
                tab_hbm.at[gidx_v.at[slot]],
                rows_v.at[pl.ds(slot * _BW, _BW)], sgs[slot],
            )

        def out_copy(s, slot):
            h, b0 = coords(s)
            return pltpu.make_async_copy(
                outt_v.at[slot, pl.ds(0, D), pl.ds(0, _BW)],
                o_hbm.at[pl.ds(h * D, D), pl.ds(b0, _BW)], sos[slot],
            )

        def fire(s, slot):
            idx_copy(s, slot).wait()
            for g in range(G):
                gidx_v[slot, pl.ds(g * _L, _L)] = raw_v[slot, pl.ds(g * _L, _L)]
            gather_copy(slot).start()

        def drain(s, slot):
            gather_copy(slot).wait()
            jbase = slot * _BW
            dvs = [lax.iota(jnp.int32, _L) + (k * _L) for k in range(D // _L)]
            sv = jnp.full((_L,), slot, jnp.int32)

            @plsc.parallel_loop(0, _BW, unroll=4)
            def _(j):
                jv = jnp.full((_L,), j, jnp.int32)
                for k in range(D // _L):
                    vals = rows_v[jbase + j, pl.ds(k * _L, _L)]
                    plsc.store_scatter(outt_v, [sv, dvs[k], jv], vals)

            out_copy(s, slot).start()

        idx_copy(0, 0).start()
        idx_copy(1, 1).start()

        @pl.loop(0, steps // 2)
        def _(o):
            for b in range(2):
                s = 2 * o + b
                fire(s, b)

                @pl.when(o < steps // 2 - 1)
                def _():
                    idx_copy(s + 2, b).start()

                prev = 1 - b
                if b == 0:
                    @pl.when(o > 1)
                    def _():
                        out_copy(2 * o - 3, prev).wait()

                    @pl.when(o > 0)
                    def _():
                        drain(2 * o - 1, prev)
                else:
                    @pl.when(o > 0)
                    def _():
                        out_copy(2 * o - 2, prev).wait()

                    drain(2 * o, prev)

        out_copy(steps - 3, 1).wait()
        drain(steps - 1, 1)
        out_copy(steps - 2, 0).wait()
        out_copy(steps - 1, 1).wait()

    y = gather_kernel(table2, tT)
    return y.reshape(H, D, B).transpose(2, 0, 1)


# speedup vs baseline: 2.3414x; 2.3414x over previous
import jax
import jax.numpy as jnp
from jax import lax
from jax.experimental import pallas as pl
from jax.experimental.pallas import tpu as pltpu
from jax.experimental.pallas import tpu_sc as plsc

_L = 16
_BW = 128
_NW = 32


def kernel(t, embeddings):
    B, H = t.shape
    V, D = embeddings.shape
    n_blocks = H * (B // _BW)
    steps = n_blocks // _NW
    assert n_blocks % _NW == 0 and steps % 2 == 0 and steps >= 4
    assert D % 8 == 0 and V % 2 == 0

    tT = t.T.astype(jnp.int32)
    table2 = jnp.concatenate([embeddings, embeddings], axis=1)
    b_tiles = B // _BW
    G = _BW // _L

    mesh = plsc.VectorSubcoreMesh(core_axis_name="core", subcore_axis_name="subcore")

    @pl.kernel(
        out_type=jax.ShapeDtypeStruct((H * D, B), embeddings.dtype),
        mesh=mesh,
        compiler_params=pltpu.CompilerParams(
            use_tc_tiling_on_sc=True, needs_layout_passes=False
        ),
        scratch_types=[
            pltpu.VMEM((2, _BW), jnp.int32),
            pltpu.VMEM((2, _BW), jnp.int32),
            pltpu.VMEM((2 * _BW, 2 * D), jnp.float32),
            pltpu.VMEM((2, D, _BW), jnp.float32),
            pltpu.SemaphoreType.DMA,
            pltpu.SemaphoreType.DMA,
            pltpu.SemaphoreType.DMA,
            pltpu.SemaphoreType.DMA,
            pltpu.SemaphoreType.DMA,
            pltpu.SemaphoreType.DMA,
        ],
    )
    def gather_kernel(tab_hbm, idx_hbm, o_hbm, raw_v, gidx_v, rows_v,
                      outt_v, si0, si1, sg0, sg1, so0, so1):
        w = lax.axis_index("subcore") * 2 + lax.axis_index("core")
        base = w * steps
        sis = (si0, si1)
        sgs = (sg0, sg1)
        sos = (so0, so1)

        def coords(s):
            blk = base + s
            return blk // b_tiles, (blk % b_tiles) * _BW

        def idx_copy(s, slot):
            h, b0 = coords(s)
            return pltpu.make_async_copy(
                idx_hbm.at[pl.ds(h, 1), pl.ds(b0, _BW)],
                raw_v.at[pl.ds(slot, 1)], sis[slot],
            )

        def gather_copy(slot):
            return pltpu.make_async_copy(
                tab_hbm.at[gidx_v.at[slot]],
                rows_v.at[pl.ds(slot * _BW, _BW)], sgs[slot],
            )

        def out_copy(s, slot):
            h, b0 = coords(s)
            return pltpu.make_async_copy(
                outt_v.at[slot],
                o_hbm.at[pl.ds(h * D, D), pl.ds(b0, _BW)], sos[slot],
            )

        def fire(s, slot):
            idx_copy(s, slot).wait()
            for g in range(G):
                gidx_v[slot, pl.ds(g * _L, _L)] = raw_v[slot, pl.ds(g * _L, _L)]
            gather_copy(slot).start()

        def drain(s, slot):
            gather_copy(slot).wait()
            jbase = slot * _BW
            iv = lax.iota(jnp.int32, _L)
            sv = jnp.full((_L,), slot, jnp.int32)
            jins = [iv + (jbase + jg * _L) for jg in range(G)]
            jouts = [iv + (jg * _L) for jg in range(G)]

            @plsc.parallel_loop(0, _L, unroll=2)
            def _(k):
                rot = (iv + k) & (_L - 1)
                dvk = [rot + (dg * _L) for dg in range(D // _L)]
                for jg in range(G):
                    for dg in range(D // _L):
                        vals = plsc.load_gather(rows_v, [jins[jg], dvk[dg]])
                        plsc.store_scatter(outt_v, [sv, dvk[dg], jouts[jg]], vals)

            out_copy(s, slot).start()

        idx_copy(0, 0).start()
        idx_copy(1, 1).start()

        @pl.loop(0, steps // 2)
        def _(o):
            for b in range(2):
                s = 2 * o + b
                fire(s, b)

                @pl.when(o < steps // 2 - 1)
                def _():
                    idx_copy(s + 2, b).start()

                prev = 1 - b
                if b == 0:
                    @pl.when(o > 1)
                    def _():
                        out_copy(2 * o - 3, prev).wait()

                    @pl.when(o > 0)
                    def _():
                        drain(2 * o - 1, prev)
                else:
                    @pl.when(o > 0)
                    def _():
                        out_copy(2 * o - 2, prev).wait()

                    drain(2 * o, prev)

        out_copy(steps - 3, 1).wait()
        drain(steps - 1, 1)
        out_copy(steps - 2, 0).wait()
        out_copy(steps - 1, 1).wait()

    y = gather_kernel(table2, tT)
    return y.reshape(H, D, B).transpose(2, 0, 1)


# speedup vs baseline: 2.3928x; 1.0220x over previous
import jax
import jax.numpy as jnp
from jax import lax
from jax.experimental import pallas as pl
from jax.experimental.pallas import tpu as pltpu
from jax.experimental.pallas import tpu_sc as plsc

_L = 16
_BW = 128
_NW = 32


def kernel(t, embeddings):
    B, H = t.shape
    V, D = embeddings.shape
    n_blocks = H * (B // _BW)
    steps = n_blocks // _NW
    assert n_blocks % _NW == 0 and steps % 2 == 0 and steps >= 4
    assert D % 8 == 0 and V % 2 == 0

    tT = t.T.astype(jnp.int32)
    table2 = jnp.concatenate([embeddings, embeddings], axis=1)
    b_tiles = B // _BW
    G = _BW // _L

    mesh = plsc.VectorSubcoreMesh(core_axis_name="core", subcore_axis_name="subcore")

    @pl.kernel(
        out_type=jax.ShapeDtypeStruct((H * D, B), embeddings.dtype),
        mesh=mesh,
        compiler_params=pltpu.CompilerParams(
            use_tc_tiling_on_sc=True, needs_layout_passes=False
        ),
        scratch_types=[
            pltpu.VMEM((2, _BW), jnp.int32),
            pltpu.VMEM((2, _BW), jnp.int32),
            pltpu.VMEM((2 * _BW, 2 * D), jnp.float32),
            pltpu.VMEM((2, D, _BW), jnp.float32),
            pltpu.SemaphoreType.DMA,
            pltpu.SemaphoreType.DMA,
            pltpu.SemaphoreType.DMA,
            pltpu.SemaphoreType.DMA,
            pltpu.SemaphoreType.DMA,
            pltpu.SemaphoreType.DMA,
        ],
    )
    def gather_kernel(tab_hbm, idx_hbm, o_hbm, raw_v, gidx_v, rows_v,
                      outt_v, si0, si1, sg0, sg1, so0, so1):
        w = lax.axis_index("subcore") * 2 + lax.axis_index("core")
        base = w * steps
        sis = (si0, si1)
        sgs = (sg0, sg1)
        sos = (so0, so1)

        def coords(s):
            blk = base + s
            return blk // b_tiles, (blk % b_tiles) * _BW

        def idx_copy(s, slot):
            h, b0 = coords(s)
            return pltpu.make_async_copy(
                idx_hbm.at[pl.ds(h, 1), pl.ds(b0, _BW)],
                raw_v.at[pl.ds(slot, 1)], sis[slot],
            )

        def gather_copy(slot):
            return pltpu.make_async_copy(
                tab_hbm.at[gidx_v.at[slot]],
                rows_v.at[pl.ds(slot * _BW, _BW)], sgs[slot],
            )

        def out_copy(s, slot):
            h, b0 = coords(s)
            return pltpu.make_async_copy(
                outt_v.at[slot],
                o_hbm.at[pl.ds(h * D, D), pl.ds(b0, _BW)], sos[slot],
            )

        def fire(s, slot):
            idx_copy(s, slot).wait()
            for g in range(G):
                gidx_v[slot, pl.ds(g * _L, _L)] = raw_v[slot, pl.ds(g * _L, _L)]
            gather_copy(slot).start()

        def drain(s, slot):
            gather_copy(slot).wait()
            jbase = slot * _BW
            iv = lax.iota(jnp.int32, _L)
            sv = jnp.full((_L,), slot, jnp.int32)
            jins = [iv + (jbase + jg * _L) for jg in range(G)]
            jouts = [iv + (jg * _L) for jg in range(G)]

            @plsc.parallel_loop(0, _L, unroll=4)
            def _(k):
                rot = (iv + k) & (_L - 1)
                dvk = [rot + (dg * _L) for dg in range(D // _L)]
                for jg in range(G):
                    for dg in range(D // _L):
                        vals = plsc.load_gather(rows_v, [jins[jg], dvk[dg]])
                        plsc.store_scatter(outt_v, [sv, dvk[dg], jouts[jg]], vals)

            out_copy(s, slot).start()

        idx_copy(0, 0).start()
        idx_copy(1, 1).start()

        @pl.loop(0, steps // 2)
        def _(o):
            for b in range(2):
                s = 2 * o + b
                fire(s, b)

                @pl.when(o < steps // 2 - 1)
                def _():
                    idx_copy(s + 2, b).start()

                prev = 1 - b
                if b == 0:
                    @pl.when(o > 1)
                    def _():
                        out_copy(2 * o - 3, prev).wait()

                    @pl.when(o > 0)
                    def _():
                        drain(2 * o - 1, prev)
                else:
                    @pl.when(o > 0)
                    def _():
                        out_copy(2 * o - 2, prev).wait()

                    drain(2 * o, prev)

        out_copy(steps - 3, 1).wait()
        drain(steps - 1, 1)
        out_copy(steps - 2, 0).wait()
        out_copy(steps - 1, 1).wait()

    y = gather_kernel(table2, tT)
    return y.reshape(H, D, B).transpose(2, 0, 1)
